# baseline (speedup 1.0000x reference)
import functools

import jax
import jax.numpy as jnp
from jax import lax
from jax.experimental import pallas as pl
from jax.experimental.pallas import tpu as pltpu
from jax.experimental.pallas import tpu_sc as plsc

_CW = 8
_W = _CW * 128
_CAP = 2048
_CCAP = 48


def kernel(attribute_value, table):
    idx = jnp.squeeze(attribute_value).astype(jnp.int32)
    (B,) = idx.shape
    V, D = table.shape

    info = plsc.get_sparse_core_info()
    NC, NS, L = info.num_cores, info.num_subcores, info.num_lanes
    NW = NC * NS
    b_per_w = B // NW
    n_vec = B // L

    tcols = (V + 127) // 128
    cpw = -(-tcols // NW)
    n_ch = -(-cpw // _CW)
    max_col = (V + 127) // 128 - _CW
    B_pad = B + NW

    mesh = plsc.VectorSubcoreMesh(core_axis_name="c", subcore_axis_name="s")

    @functools.partial(
        pl.kernel,
        mesh=mesh,
        out_type=jax.ShapeDtypeStruct((B_pad, 128), jnp.float32),
        compiler_params=pltpu.CompilerParams(needs_layout_passes=False),
        scratch_types=[
            pltpu.VMEM((B,), jnp.int32),
            pltpu.VMEM((_CAP,), jnp.int32),
            pltpu.VMEM((_CAP,), jnp.int32),
            pltpu.VMEM((2, D // 8, 8, _W), jnp.float32),
            pltpu.VMEM((2, _CCAP,), jnp.int32),
            pltpu.VMEM((2, _CCAP,), jnp.int32),
            pltpu.VMEM((2, _CCAP, 128), jnp.float32),
            pltpu.SemaphoreType.DMA((2,)),
            pltpu.SemaphoreType.DMA((2,)),
        ],
    )
    def scan_kernel(
        table_hbm, idx_hbm, stage_hbm, idx_all, hpos, hval, bufs, cpos, clan,
        cvals, fsem, ssem
    ):
        wid = lax.axis_index("s") * NC + lax.axis_index("c")
        lo_col = wid * cpw
        hi_col = lax.min(lo_col + cpw, tcols)
        c0 = lax.iota(jnp.int32, L)
        n_tr = D // 8

        def chunk_start(c):
            return lax.min(lo_col + c * _CW, max_col) * 128

        def fetch(c, b):
            start = pl.multiple_of(chunk_start(c), 128)
            for t in range(n_tr):
                pltpu.async_copy(
                    view3.at[t, :, pl.ds(start, _W)],
                    bufs.at[b, t],
                    fsem.at[b],
                )

        view3 = table_hbm.reshape(n_tr, 8, V)

        pltpu.sync_copy(idx_hbm, idx_all)
        fetch(0, 0)
        fetch(1, 1)

        lo_v = lax.broadcast(lo_col, (L,))
        hi_v = lax.broadcast(hi_col, (L,))

        def p1(b, nh):
            r = idx_all[pl.ds(b * L, L)]
            j = r >> 7
            m = (j >= lo_v) & (j < hi_v)
            plsc.store_compressed(hpos.at[pl.ds(nh, L)], b * L + c0, mask=m)
            plsc.store_compressed(hval.at[pl.ds(nh, L)], r, mask=m)
            return nh + plsc.all_reduce_population_count(m)[0]

        nh = lax.fori_loop(0, n_vec, p1, 0)
        nh_v = lax.broadcast(nh, (L,))
        n_hv = (nh + L - 1) // L
        dump = lax.broadcast(B + wid, (L,))
        zeros = lax.broadcast(0, (L,))

        t_lo = c0 >> 3
        s_ids = c0 & 7

        def do_chunk(c, _):
            b = lax.rem(c, 2)

            @pl.when(c >= 2)
            def _():
                pltpu.make_async_copy(
                    cvals.at[b], stage_hbm.at[cpos.at[b]], ssem.at[b]
                ).wait()
            for q in range(_CCAP // L):
                cpos[b, pl.ds(q * L, L)] = dump
                clan[b, pl.ds(q * L, L)] = zeros
            pltpu.make_async_copy(
                view3.at[:, :, pl.ds(0, _W)], bufs.at[b], fsem.at[b]
            ).wait()
            start = chunk_start(c)
            start_v = lax.broadcast(start, (L,))

            def pc(h, mh):
                hp = hpos[pl.ds(h * L, L)]
                hr = hval[pl.ds(h * L, L)]
                l = hr - start_v
                m = (
                    (l >= 0)
                    & (l < _W)
                    & ((h * L + c0) < nh_v)
                )
                plsc.store_compressed(cpos.at[b].at[pl.ds(mh, L)], hp, mask=m)
                plsc.store_compressed(clan.at[b].at[pl.ds(mh, L)], l, mask=m)
                return mh + plsc.all_reduce_population_count(m)[0]

            mh = lax.fori_loop(0, n_hv, pc, 0)

            def px(h, _c):
                lvecs = clan[b, pl.ds(h * L, L)]
                for lane in range(L):
                    lv = lax.broadcast(lvecs[lane], (L,))
                    g0 = plsc.load_gather(bufs.at[b], [t_lo, s_ids, lv])
                    g1 = plsc.load_gather(bufs.at[b], [t_lo + 2, s_ids, lv])
                    row = h * L + lane
                    cvals[b, row, pl.ds(0, L)] = g0
                    cvals[b, row, pl.ds(L, L)] = g1
                return _c

            lax.fori_loop(0, (mh + L - 1) // L, px, 0)

            @pl.when(c + 2 < n_ch)
            def _():
                fetch(c + 2, b)

            pltpu.async_copy(cvals.at[b], stage_hbm.at[cpos.at[b]], ssem.at[b])
            return 0

        lax.fori_loop(0, n_ch, do_chunk, 0)
        for b in range(2):
            pltpu.make_async_copy(
                cvals.at[b], stage_hbm.at[cpos.at[b]], ssem.at[b]
            ).wait()

    @functools.partial(
        pl.kernel,
        mesh=mesh,
        out_type=jax.ShapeDtypeStruct((D, B), jnp.float32),
        compiler_params=pltpu.CompilerParams(needs_layout_passes=False),
        scratch_types=[
            pltpu.VMEM((b_per_w, 128), jnp.float32),
            pltpu.VMEM((D, b_per_w), jnp.float32),
        ],
    )
    def transpose_kernel(stage_hbm, out_hbm, stag_v, rows_v):
        wid = lax.axis_index("s") * NC + lax.axis_index("c")
        base = pl.multiple_of(wid * b_per_w, b_per_w)
        pltpu.sync_copy(stage_hbm.at[pl.ds(base, b_per_w)], stag_v)
        c0 = lax.iota(jnp.int32, L)

        def blk(q, _c):
            ivec = q * L + c0
            for c in range(D):
                cv = lax.broadcast(c, (L,))
                g = plsc.load_gather(stag_v, [ivec, cv])
                rows_v[c, pl.ds(q * L, L)] = g
            return _c

        lax.fori_loop(0, b_per_w // L, blk, 0)
        pltpu.sync_copy(rows_v, out_hbm.at[:, pl.ds(base, b_per_w)])

    stage = scan_kernel(table.T, idx)
    outT = transpose_kernel(stage)
    return outT.T
